# baseline (speedup 1.0000x reference)
import functools

import jax
import jax.numpy as jnp
from jax import lax
from jax.experimental import pallas as pl
from jax.experimental.pallas import tpu as pltpu
from jax.experimental.pallas import tpu_sc as plsc

N = 10000
E = 320000
R = 3
F = 128
H = 64
O = 64
B = 1024

NC = 2
NS = 16
NW = NC * NS
CH = 128
EROWS = 2560
EP = EROWS * CH
WROWS = EROWS // NW
W0 = 128
W1 = 32
ROWS0 = W0 * NS
NP = 10112
SLICE = NP // NS
BW = B // NW

_mesh = plsc.VectorSubcoreMesh(core_axis_name="c", subcore_axis_name="s")
_sc_params = pltpu.CompilerParams(use_tc_tiling_on_sc=False)


def _deg_body(d0, d1, d2, z16, ones_hbm, out, idx_v, ones_v, sem, deg_sh):
    c = lax.axis_index("c")
    s = lax.axis_index("s")
    wrow = (c * NS + s) * WROWS
    pltpu.sync_copy(ones_hbm, ones_v)
    for r, dref in enumerate((d0, d1, d2)):
        pltpu.sync_copy(z16.at[pl.ds(s * SLICE, SLICE)],
                        deg_sh.at[pl.ds(s * SLICE, SLICE)])
        pltpu.sync_copy(dref.at[pl.ds(wrow, WROWS)], idx_v)
        plsc.subcore_barrier()

        def step(j, carry):
            pltpu.sync_copy(ones_v, deg_sh.at[idx_v.at[j]], add=True)
            return carry

        lax.fori_loop(0, WROWS, step, 0)
        plsc.subcore_barrier()
        pltpu.sync_copy(deg_sh.at[pl.ds(s * SLICE, SLICE)],
                        out.at[c, r, pl.ds(s * SLICE, SLICE)])


_sc_deg = functools.partial(
    pl.kernel,
    out_type=jax.ShapeDtypeStruct((NC, R, NP, 16), jnp.float32),
    mesh=_mesh,
    compiler_params=_sc_params,
    scratch_types=[
        pltpu.VMEM((WROWS, CH), jnp.int32),
        pltpu.VMEM((CH, 16), jnp.float32),
        pltpu.SemaphoreType.DMA,
        pltpu.VMEM_SHARED((NP, 16), jnp.float32),
    ],
)(_deg_body)


def _scat_body(href, sref, dref, z64, out,
               idx_s, idx_d, rows_a, rows_b, rows_c, rows_d,
               sem_a, sem_b, sem_c, sem_d, acc_sh):
    c = lax.axis_index("c")
    s = lax.axis_index("s")
    wrow = jnp.where(c == 0, s * W0, ROWS0 + s * W1)
    ng = jnp.where(c == 0, W0 // 4, W1 // 4)
    WMIN = min(W0, W1)
    pltpu.sync_copy(z64.at[pl.ds(s * SLICE, SLICE)],
                    acc_sh.at[pl.ds(s * SLICE, SLICE)])
    pltpu.sync_copy(sref.at[pl.ds(wrow, WMIN)], idx_s.at[pl.ds(0, WMIN)])
    pltpu.sync_copy(dref.at[pl.ds(wrow, WMIN)], idx_d.at[pl.ds(0, WMIN)])

    @pl.when(c == 0)
    def _():
        pltpu.sync_copy(sref.at[pl.ds(wrow + WMIN, W0 - WMIN)],
                        idx_s.at[pl.ds(WMIN, W0 - WMIN)])
        pltpu.sync_copy(dref.at[pl.ds(wrow + WMIN, W0 - WMIN)],
                        idx_d.at[pl.ds(WMIN, W0 - WMIN)])

    plsc.subcore_barrier()

    bufs = ((rows_a, sem_a), (rows_b, sem_b), (rows_c, sem_c),
            (rows_d, sem_d))
    for k, (rv, sm) in enumerate(bufs):
        pltpu.async_copy(href.at[idx_s.at[k]], rv, sm)

    def grp(g, carry):
        j0 = 4 * g
        for k, (rv, sm) in enumerate(bufs):
            pltpu.make_async_copy(href.at[idx_s.at[j0 + k]], rv,
                                  sm).wait()
            pltpu.sync_copy(rv, acc_sh.at[idx_d.at[j0 + k]], add=True)

            @pl.when(g < ng - 1)
            def _():
                pltpu.async_copy(href.at[idx_s.at[j0 + k + 4]], rv, sm)

        return carry

    lax.fori_loop(0, ng, grp, 0)
    plsc.subcore_barrier()
    pltpu.sync_copy(acc_sh.at[pl.ds(s * SLICE, SLICE)],
                    out.at[c, pl.ds(s * SLICE, SLICE)])


_sc_scatter = functools.partial(
    pl.kernel,
    out_type=jax.ShapeDtypeStruct((NC, NP, O), jnp.float32),
    mesh=_mesh,
    compiler_params=_sc_params,
    scratch_types=[
        pltpu.VMEM((max(W0, W1), CH), jnp.int32),
        pltpu.VMEM((max(W0, W1), CH), jnp.int32),
        pltpu.VMEM((CH, O), jnp.float32),
        pltpu.VMEM((CH, O), jnp.float32),
        pltpu.VMEM((CH, O), jnp.float32),
        pltpu.VMEM((CH, O), jnp.float32),
        pltpu.SemaphoreType.DMA,
        pltpu.SemaphoreType.DMA,
        pltpu.SemaphoreType.DMA,
        pltpu.SemaphoreType.DMA,
        pltpu.VMEM_SHARED((NP, O), jnp.float32),
    ],
)(_scat_body)


def _gather_body(x0, x1, x2, bn, out, bidx_v, rows_v, sem):
    c = lax.axis_index("c")
    s = lax.axis_index("s")
    base = (c * NS + s) * BW
    pltpu.sync_copy(bn.at[pl.ds(base, BW)], bidx_v)
    for r, xref in enumerate((x0, x1, x2)):
        pltpu.async_copy(xref.at[bidx_v], rows_v, sem).wait()
        pltpu.sync_copy(rows_v, out.at[r, pl.ds(base, BW)])


_sc_gather = functools.partial(
    pl.kernel,
    out_type=jax.ShapeDtypeStruct((R, B, O), jnp.float32),
    mesh=_mesh,
    compiler_params=_sc_params,
    scratch_types=[
        pltpu.VMEM((BW,), jnp.int32),
        pltpu.VMEM((BW, O), jnp.float32),
        pltpu.SemaphoreType.DMA,
    ],
)(_gather_body)


NB = 1000
NBK = N // NB


def _mm1_body(x_ref, w_ref, degp_ref, h_ref, hs_ref, dv_ref):
    h = jnp.dot(x_ref[...], w_ref[...], preferred_element_type=jnp.float32)
    deg = degp_ref[0, :, :1] + degp_ref[1, :, :1] + 1.0
    dinv = lax.rsqrt(deg)
    h_ref[...] = h
    hs_ref[...] = h * dinv
    dv_ref[...] = jnp.broadcast_to(dinv, (NB, H))


def _tc_mm1(x, w1r, degpr):
    return pl.pallas_call(
        _mm1_body,
        grid=(NBK,),
        in_specs=[
            pl.BlockSpec((NB, F), lambda i: (i, 0)),
            pl.BlockSpec((F, H), lambda i: (0, 0)),
            pl.BlockSpec((NC, NB, 16), lambda i: (0, i, 0)),
        ],
        out_specs=[
            pl.BlockSpec((NB, H), lambda i: (i, 0)),
            pl.BlockSpec((NB, H), lambda i: (i, 0)),
            pl.BlockSpec((NB, H), lambda i: (i, 0)),
        ],
        out_shape=[
            jax.ShapeDtypeStruct((N, H), jnp.float32),
            jax.ShapeDtypeStruct((N, H), jnp.float32),
            jax.ShapeDtypeStruct((N, H), jnp.float32),
        ],
    )(x, w1r, degpr)


def _s1_body(accp_ref, h_ref, dv_ref, b_ref, t_ref, ps_ref):
    acc = accp_ref[0] + accp_ref[1]
    dinv = dv_ref[...]
    t = acc * dinv + h_ref[...] * (dinv * dinv) + b_ref[...]
    t_ref[...] = t
    s1 = jnp.sum(t, axis=0, keepdims=True)
    s2 = jnp.sum(t * t, axis=0, keepdims=True)
    ps_ref[0] = jnp.concatenate(
        [s1, s2, jnp.zeros((6, H), jnp.float32)], axis=0)


def _s2_body(t_ref, ps_ref, dv_ref, g_ref, be_ref, w2_ref, out1_ref,
             out2_ref, last):
    ps = ps_ref[...]
    m = jnp.sum(ps[:, 0, :], axis=0, keepdims=True) * (1.0 / N)
    ex2 = jnp.sum(ps[:, 1, :], axis=0, keepdims=True) * (1.0 / N)
    v = ex2 - m * m
    t = t_ref[...]
    xn = (t - m) * lax.rsqrt(v + 1e-5) * g_ref[...] + be_ref[...]
    x1 = jnp.maximum(xn, 0.0)
    if last:
        out1_ref[...] = x1
    else:
        dinv = dv_ref[...]
        h2 = jnp.dot(x1, w2_ref[...], preferred_element_type=jnp.float32)
        out1_ref[...] = h2
        out2_ref[...] = h2 * dinv


def _tc_stage(accp, h, dv, br, gr, ber, w2r, last):
    t, ps = pl.pallas_call(
        _s1_body,
        grid=(NBK,),
        in_specs=[
            pl.BlockSpec((NC, NB, H), lambda i: (0, i, 0)),
            pl.BlockSpec((NB, H), lambda i: (i, 0)),
            pl.BlockSpec((NB, H), lambda i: (i, 0)),
            pl.BlockSpec((1, H), lambda i: (0, 0)),
        ],
        out_specs=[
            pl.BlockSpec((NB, H), lambda i: (i, 0)),
            pl.BlockSpec((1, 8, H), lambda i: (i, 0, 0)),
        ],
        out_shape=[
            jax.ShapeDtypeStruct((N, H), jnp.float32),
            jax.ShapeDtypeStruct((NBK, 8, H), jnp.float32),
        ],
    )(accp, h, dv, br.reshape(1, H))
    body = functools.partial(_s2_body, last=last,
                             **({"out2_ref": None} if last else {}))
    if last:
        out_specs = pl.BlockSpec((NB, H), lambda i: (i, 0))
        out_shape = jax.ShapeDtypeStruct((N, H), jnp.float32)
    else:
        out_specs = [pl.BlockSpec((NB, H), lambda i: (i, 0)),
                     pl.BlockSpec((NB, H), lambda i: (i, 0))]
        out_shape = [jax.ShapeDtypeStruct((N, H), jnp.float32),
                     jax.ShapeDtypeStruct((N, H), jnp.float32)]
    res = pl.pallas_call(
        body,
        grid=(NBK,),
        in_specs=[
            pl.BlockSpec((NB, H), lambda i: (i, 0)),
            pl.BlockSpec((NBK, 8, H), lambda i: (0, 0, 0)),
            pl.BlockSpec((NB, H), lambda i: (i, 0)),
            pl.BlockSpec((1, H), lambda i: (0, 0)),
            pl.BlockSpec((1, H), lambda i: (0, 0)),
            pl.BlockSpec((H, O), lambda i: (0, 0)),
        ],
        out_specs=out_specs,
        out_shape=out_shape,
    )(t, ps, dv, gr.reshape(1, H), ber.reshape(1, H), w2r)
    return res


def _final_body(m_ref, w_ref, b_ref, u_ref, out_ref):
    ls = []
    scores = []
    for r in range(R):
        x = m_ref[r]
        mx = jnp.max(x, axis=1, keepdims=True)
        sh = x - mx
        l = sh - jnp.log(jnp.sum(jnp.exp(sh), axis=1, keepdims=True))
        ls.append(l)
        v = jnp.tanh(
            jnp.dot(l, w_ref[...], preferred_element_type=jnp.float32)
            + b_ref[...])
        scores.append(jnp.sum(v * u_ref[...], axis=1, keepdims=True))
    s = jnp.concatenate(scores, axis=1)
    smx = jnp.max(s, axis=1, keepdims=True)
    es = jnp.exp(s - smx)
    alpha = es / jnp.sum(es, axis=1, keepdims=True)
    out = ls[0] * alpha[:, 0:1]
    for r in range(1, R):
        out = out + ls[r] * alpha[:, r:r + 1]
    out_ref[...] = out


def _tc_final(m, w_omega, b_omega, u_omega):
    return pl.pallas_call(
        _final_body,
        out_shape=jax.ShapeDtypeStruct((B, O), jnp.float32),
    )(m, w_omega, b_omega.reshape(1, H), u_omega.reshape(1, H))


def kernel(features, multi_r_data, batch_nodes, W1, b1, g1, be1, W2, b2, g2,
           be2, w_omega, b_omega, u_omega):
    pad_s = jnp.zeros((EP - E,), jnp.int32)
    pad_d = jnp.full((EP - E,), N, jnp.int32)
    srcs, dsts = [], []
    for r in range(R):
        srcs.append(jnp.concatenate([multi_r_data[r, 0], pad_s])
                    .reshape(EROWS, CH))
        dsts.append(jnp.concatenate([multi_r_data[r, 1], pad_d])
                    .reshape(EROWS, CH))
    z16 = jnp.zeros((NP, 16), jnp.float32)
    z64 = jnp.zeros((NP, O), jnp.float32)
    ones16 = jnp.ones((CH, 16), jnp.float32)

    degp = _sc_deg(dsts[0], dsts[1], dsts[2], z16, ones16)
    h1, hs1, dv, acc1, h2, hs2, acc2, x2 = {}, {}, {}, {}, {}, {}, {}, {}
    for r in range(R):
        h1[r], hs1[r], dv[r] = _tc_mm1(features, W1[r], degp[:, r])
    for r in range(R):
        acc1[r] = _sc_scatter(hs1[r], srcs[r], dsts[r], z64)
    for r in range(R):
        h2[r], hs2[r] = _tc_stage(acc1[r], h1[r], dv[r], b1[r], g1[r],
                                  be1[r], W2[r], last=False)
    for r in range(R):
        acc2[r] = _sc_scatter(hs2[r], srcs[r], dsts[r], z64)
    for r in range(R):
        x2[r] = _tc_stage(acc2[r], h2[r], dv[r], b2[r], g2[r], be2[r],
                          W2[r], last=True)
    m = _sc_gather(x2[0], x2[1], x2[2], batch_nodes)
    return _tc_final(m, w_omega, b_omega, u_omega)


# speedup vs baseline: 1.0578x; 1.0578x over previous
import functools

import jax
import jax.numpy as jnp
from jax import lax
from jax.experimental import pallas as pl
from jax.experimental.pallas import tpu as pltpu
from jax.experimental.pallas import tpu_sc as plsc

N = 10000
E = 320000
R = 3
F = 128
H = 64
O = 64
B = 1024

NC = 2
NS = 16
NW = NC * NS
CH = 128
EROWS = 2560
EP = EROWS * CH
WROWS = EROWS // NW
W0 = 128
W1 = 32
ROWS0 = W0 * NS
NP = 10112
SLICE = NP // NS
BW = B // NW

_mesh = plsc.VectorSubcoreMesh(core_axis_name="c", subcore_axis_name="s")
_sc_params = pltpu.CompilerParams(use_tc_tiling_on_sc=False)


def _deg_body(d0, d1, d2, z16, ones_hbm, out, idx_v, ones_v, sem, deg_sh):
    c = lax.axis_index("c")
    s = lax.axis_index("s")
    wrow = (c * NS + s) * WROWS
    pltpu.sync_copy(ones_hbm, ones_v)
    for r, dref in enumerate((d0, d1, d2)):
        pltpu.sync_copy(z16.at[pl.ds(s * SLICE, SLICE)],
                        deg_sh.at[pl.ds(s * SLICE, SLICE)])
        pltpu.sync_copy(dref.at[pl.ds(wrow, WROWS)], idx_v)
        plsc.subcore_barrier()

        def step(j, carry):
            pltpu.sync_copy(ones_v, deg_sh.at[idx_v.at[j]], add=True)
            return carry

        lax.fori_loop(0, WROWS, step, 0)
        plsc.subcore_barrier()
        pltpu.sync_copy(deg_sh.at[pl.ds(s * SLICE, SLICE)],
                        out.at[c, r, pl.ds(s * SLICE, SLICE)])


_sc_deg = functools.partial(
    pl.kernel,
    out_type=jax.ShapeDtypeStruct((NC, R, NP, 16), jnp.float32),
    mesh=_mesh,
    compiler_params=_sc_params,
    scratch_types=[
        pltpu.VMEM((WROWS, CH), jnp.int32),
        pltpu.VMEM((CH, 16), jnp.float32),
        pltpu.SemaphoreType.DMA,
        pltpu.VMEM_SHARED((NP, 16), jnp.float32),
    ],
)(_deg_body)


def _scat_body(h0, h1, h2, s0, s1, s2, d0, d1, d2, z64, out,
               idx_s, idx_d, rows_a, rows_b, rows_c, rows_d,
               sem_a, sem_b, sem_c, sem_d, acc_sh):
    c = lax.axis_index("c")
    s = lax.axis_index("s")
    wrow = jnp.where(c == 0, s * W0, ROWS0 + s * W1)
    ng = jnp.where(c == 0, W0 // 4, W1 // 4)
    WMIN = min(W0, W1)
    for r, (href, sref, dref) in enumerate(((h0, s0, d0), (h1, s1, d1),
                                            (h2, s2, d2))):
        pltpu.sync_copy(z64.at[pl.ds(s * SLICE, SLICE)],
                        acc_sh.at[pl.ds(s * SLICE, SLICE)])
        pltpu.sync_copy(sref.at[pl.ds(wrow, WMIN)], idx_s.at[pl.ds(0, WMIN)])
        pltpu.sync_copy(dref.at[pl.ds(wrow, WMIN)], idx_d.at[pl.ds(0, WMIN)])

        @pl.when(c == 0)
        def _():
            pltpu.sync_copy(sref.at[pl.ds(wrow + WMIN, W0 - WMIN)],
                            idx_s.at[pl.ds(WMIN, W0 - WMIN)])
            pltpu.sync_copy(dref.at[pl.ds(wrow + WMIN, W0 - WMIN)],
                            idx_d.at[pl.ds(WMIN, W0 - WMIN)])

        plsc.subcore_barrier()

        bufs = ((rows_a, sem_a), (rows_b, sem_b), (rows_c, sem_c),
                (rows_d, sem_d))
        for k, (rv, sm) in enumerate(bufs):
            pltpu.async_copy(href.at[idx_s.at[k]], rv, sm)

        def grp(g, carry):
            j0 = 4 * g
            for k, (rv, sm) in enumerate(bufs):
                pltpu.make_async_copy(href.at[idx_s.at[j0 + k]], rv,
                                      sm).wait()
                pltpu.sync_copy(rv, acc_sh.at[idx_d.at[j0 + k]], add=True)

                @pl.when(g < ng - 1)
                def _():
                    pltpu.async_copy(href.at[idx_s.at[j0 + k + 4]], rv, sm)

            return carry

        lax.fori_loop(0, ng, grp, 0)
        plsc.subcore_barrier()
        pltpu.sync_copy(acc_sh.at[pl.ds(s * SLICE, SLICE)],
                        out.at[c, r, pl.ds(s * SLICE, SLICE)])


_sc_scatter = functools.partial(
    pl.kernel,
    out_type=jax.ShapeDtypeStruct((NC, R, NP, O), jnp.float32),
    mesh=_mesh,
    compiler_params=_sc_params,
    scratch_types=[
        pltpu.VMEM((max(W0, W1), CH), jnp.int32),
        pltpu.VMEM((max(W0, W1), CH), jnp.int32),
        pltpu.VMEM((CH, O), jnp.float32),
        pltpu.VMEM((CH, O), jnp.float32),
        pltpu.VMEM((CH, O), jnp.float32),
        pltpu.VMEM((CH, O), jnp.float32),
        pltpu.SemaphoreType.DMA,
        pltpu.SemaphoreType.DMA,
        pltpu.SemaphoreType.DMA,
        pltpu.SemaphoreType.DMA,
        pltpu.VMEM_SHARED((NP, O), jnp.float32),
    ],
)(_scat_body)


def _gather_body(x0, x1, x2, bn, out, bidx_v, rows_v, sem):
    c = lax.axis_index("c")
    s = lax.axis_index("s")
    base = (c * NS + s) * BW
    pltpu.sync_copy(bn.at[pl.ds(base, BW)], bidx_v)
    for r, xref in enumerate((x0, x1, x2)):
        pltpu.async_copy(xref.at[bidx_v], rows_v, sem).wait()
        pltpu.sync_copy(rows_v, out.at[r, pl.ds(base, BW)])


_sc_gather = functools.partial(
    pl.kernel,
    out_type=jax.ShapeDtypeStruct((R, B, O), jnp.float32),
    mesh=_mesh,
    compiler_params=_sc_params,
    scratch_types=[
        pltpu.VMEM((BW,), jnp.int32),
        pltpu.VMEM((BW, O), jnp.float32),
        pltpu.SemaphoreType.DMA,
    ],
)(_gather_body)


NB = 1000
NBK = N // NB


def _mm1_body(x_ref, w_ref, degp_ref, h_ref, hs_ref, dv_ref):
    h = jnp.dot(x_ref[...], w_ref[0], preferred_element_type=jnp.float32)
    deg = degp_ref[0, 0, :, :1] + degp_ref[1, 0, :, :1] + 1.0
    dinv = lax.rsqrt(deg)
    h_ref[0] = h
    hs_ref[0] = h * dinv
    dv_ref[0] = jnp.broadcast_to(dinv, (NB, H))


def _tc_mm1(x, w1, degp):
    return pl.pallas_call(
        _mm1_body,
        grid=(R, NBK),
        in_specs=[
            pl.BlockSpec((NB, F), lambda r, i: (i, 0)),
            pl.BlockSpec((1, F, H), lambda r, i: (r, 0, 0)),
            pl.BlockSpec((NC, 1, NB, 16), lambda r, i: (0, r, i, 0)),
        ],
        out_specs=[
            pl.BlockSpec((1, NB, H), lambda r, i: (r, i, 0)),
            pl.BlockSpec((1, NB, H), lambda r, i: (r, i, 0)),
            pl.BlockSpec((1, NB, H), lambda r, i: (r, i, 0)),
        ],
        out_shape=[
            jax.ShapeDtypeStruct((R, N, H), jnp.float32),
            jax.ShapeDtypeStruct((R, N, H), jnp.float32),
            jax.ShapeDtypeStruct((R, N, H), jnp.float32),
        ],
    )(x, w1, degp)


def _s1_body(accp_ref, h_ref, dv_ref, b_ref, t_ref, ps_ref):
    acc = accp_ref[0, 0, :, :] + accp_ref[1, 0, :, :]
    dinv = dv_ref[0]
    t = acc * dinv + h_ref[0] * (dinv * dinv) + b_ref[0]
    t_ref[0] = t
    s1 = jnp.sum(t, axis=0, keepdims=True)
    s2 = jnp.sum(t * t, axis=0, keepdims=True)
    ps_ref[0, 0] = jnp.concatenate(
        [s1, s2, jnp.zeros((6, H), jnp.float32)], axis=0)


def _s2_body(t_ref, ps_ref, dv_ref, g_ref, be_ref, w2_ref, out1_ref,
             out2_ref, last):
    ps = ps_ref[0]
    m = jnp.sum(ps[:, 0, :], axis=0, keepdims=True) * (1.0 / N)
    ex2 = jnp.sum(ps[:, 1, :], axis=0, keepdims=True) * (1.0 / N)
    v = ex2 - m * m
    t = t_ref[0]
    xn = (t - m) * lax.rsqrt(v + 1e-5) * g_ref[0] + be_ref[0]
    x1 = jnp.maximum(xn, 0.0)
    if last:
        out1_ref[0] = x1
    else:
        dinv = dv_ref[0]
        h2 = jnp.dot(x1, w2_ref[0], preferred_element_type=jnp.float32)
        out1_ref[0] = h2
        out2_ref[0] = h2 * dinv


def _tc_stage(accp, h, dv, b, g, be, w2, last):
    t, ps = pl.pallas_call(
        _s1_body,
        grid=(R, NBK),
        in_specs=[
            pl.BlockSpec((NC, 1, NB, H), lambda r, i: (0, r, i, 0)),
            pl.BlockSpec((1, NB, H), lambda r, i: (r, i, 0)),
            pl.BlockSpec((1, NB, H), lambda r, i: (r, i, 0)),
            pl.BlockSpec((1, 1, H), lambda r, i: (r, 0, 0)),
        ],
        out_specs=[
            pl.BlockSpec((1, NB, H), lambda r, i: (r, i, 0)),
            pl.BlockSpec((1, 1, 8, H), lambda r, i: (r, i, 0, 0)),
        ],
        out_shape=[
            jax.ShapeDtypeStruct((R, N, H), jnp.float32),
            jax.ShapeDtypeStruct((R, NBK, 8, H), jnp.float32),
        ],
    )(accp, h, dv, b.reshape(R, 1, H))
    body = functools.partial(_s2_body, last=last,
                             **({"out2_ref": None} if last else {}))
    if last:
        out_specs = pl.BlockSpec((1, NB, H), lambda r, i: (r, i, 0))
        out_shape = jax.ShapeDtypeStruct((R, N, H), jnp.float32)
    else:
        out_specs = [pl.BlockSpec((1, NB, H), lambda r, i: (r, i, 0)),
                     pl.BlockSpec((1, NB, H), lambda r, i: (r, i, 0))]
        out_shape = [jax.ShapeDtypeStruct((R, N, H), jnp.float32),
                     jax.ShapeDtypeStruct((R, N, H), jnp.float32)]
    res = pl.pallas_call(
        body,
        grid=(R, NBK),
        in_specs=[
            pl.BlockSpec((1, NB, H), lambda r, i: (r, i, 0)),
            pl.BlockSpec((1, NBK, 8, H), lambda r, i: (r, 0, 0, 0)),
            pl.BlockSpec((1, NB, H), lambda r, i: (r, i, 0)),
            pl.BlockSpec((1, 1, H), lambda r, i: (r, 0, 0)),
            pl.BlockSpec((1, 1, H), lambda r, i: (r, 0, 0)),
            pl.BlockSpec((1, H, O), lambda r, i: (r, 0, 0)),
        ],
        out_specs=out_specs,
        out_shape=out_shape,
    )(t, ps, dv, g.reshape(R, 1, H), be.reshape(R, 1, H), w2)
    return res


def _final_body(m_ref, w_ref, b_ref, u_ref, out_ref):
    ls = []
    scores = []
    for r in range(R):
        x = m_ref[r]
        mx = jnp.max(x, axis=1, keepdims=True)
        sh = x - mx
        l = sh - jnp.log(jnp.sum(jnp.exp(sh), axis=1, keepdims=True))
        ls.append(l)
        v = jnp.tanh(
            jnp.dot(l, w_ref[...], preferred_element_type=jnp.float32)
            + b_ref[...])
        scores.append(jnp.sum(v * u_ref[...], axis=1, keepdims=True))
    s = jnp.concatenate(scores, axis=1)
    smx = jnp.max(s, axis=1, keepdims=True)
    es = jnp.exp(s - smx)
    alpha = es / jnp.sum(es, axis=1, keepdims=True)
    out = ls[0] * alpha[:, 0:1]
    for r in range(1, R):
        out = out + ls[r] * alpha[:, r:r + 1]
    out_ref[...] = out


def _tc_final(m, w_omega, b_omega, u_omega):
    return pl.pallas_call(
        _final_body,
        out_shape=jax.ShapeDtypeStruct((B, O), jnp.float32),
    )(m, w_omega, b_omega.reshape(1, H), u_omega.reshape(1, H))


def kernel(features, multi_r_data, batch_nodes, W1, b1, g1, be1, W2, b2, g2,
           be2, w_omega, b_omega, u_omega):
    pad_s = jnp.zeros((EP - E,), jnp.int32)
    pad_d = jnp.full((EP - E,), N, jnp.int32)
    srcs, dsts = [], []
    for r in range(R):
        srcs.append(jnp.concatenate([multi_r_data[r, 0], pad_s])
                    .reshape(EROWS, CH))
        dsts.append(jnp.concatenate([multi_r_data[r, 1], pad_d])
                    .reshape(EROWS, CH))
    z16 = jnp.zeros((NP, 16), jnp.float32)
    z64 = jnp.zeros((NP, O), jnp.float32)
    ones16 = jnp.ones((CH, 16), jnp.float32)

    degp = _sc_deg(dsts[0], dsts[1], dsts[2], z16, ones16)
    h1, hs1, dv = _tc_mm1(features, W1, degp)
    acc1 = _sc_scatter(hs1[0], hs1[1], hs1[2], srcs[0], srcs[1], srcs[2],
                       dsts[0], dsts[1], dsts[2], z64)
    h2, hs2 = _tc_stage(acc1, h1, dv, b1, g1, be1, W2, last=False)
    acc2 = _sc_scatter(hs2[0], hs2[1], hs2[2], srcs[0], srcs[1], srcs[2],
                       dsts[0], dsts[1], dsts[2], z64)
    x2 = _tc_stage(acc2, h2, dv, b2, g2, be2, W2, last=True)
    m = _sc_gather(x2[0], x2[1], x2[2], batch_nodes)
    return _tc_final(m, w_omega, b_omega, u_omega)
